# baseline (speedup 1.0000x reference)
import jax
import jax.numpy as jnp
from jax.experimental import pallas as pl
from jax.experimental.pallas import tpu as pltpu

B, S, DE, HIDDEN = 8, 1024, 4, 256
BASE_CHUNK = 64
MAX_SEQ_LEN = 512
TS = 256
K = S // TS


def _chunk_kernel(edge_ref, node_ref, mask_ref, wn_ref, bn_ref, we1_ref,
                  be1_ref, we2_ref, be2_ref, wia_ref, wib_ref, bi_ref,
                  out_ref):
    k = pl.program_id(1)

    e = edge_ref[0]
    r1 = jnp.sum(e, axis=1)
    rowsum = jnp.sum(r1, axis=1, keepdims=True)
    es = rowsum * (1.0 / (S * DE))

    node_enc = (jnp.dot(node_ref[0], wn_ref[...],
                        preferred_element_type=jnp.float32)
                + bn_ref[...])
    h = jnp.maximum(es * we1_ref[...] + be1_ref[...], 0.0)
    edge_enc = (jnp.dot(h, we2_ref[...],
                        preferred_element_type=jnp.float32)
                + be2_ref[...])

    logit = (jnp.sum(node_enc * wia_ref[...], axis=1, keepdims=True)
             + jnp.sum(edge_enc * wib_ref[...], axis=1, keepdims=True)
             + bi_ref[0, 0])
    sig = jax.nn.sigmoid(logit)
    m = mask_ref[0, 0]
    partial = jnp.dot(m, sig, preferred_element_type=jnp.float32)

    @pl.when(k == 0)
    def _init():
        out_ref[...] = jnp.zeros_like(out_ref)

    out_ref[...] += partial[0, 0]

    @pl.when(k == K - 1)
    def _finalize():
        acc = out_ref[...]
        cs = acc * (float(BASE_CHUNK) / float(S))
        cs = jnp.clip(cs, 32.0, 128.0)
        bad = (cs < 1.0) | ~jnp.isfinite(cs)
        out_ref[...] = jnp.where(bad, float(BASE_CHUNK), cs)


def kernel(node_features, edge_features, sequence_mask, W_node, b_node,
           W_e1, b_e1, W_e2, b_e2, W_imp, b_imp):
    edge_r = edge_features.reshape(B, S, (S * DE) // 128, 128)
    mask_r = sequence_mask.reshape(B, K, 1, TS)
    bn = b_node.reshape(1, 32)
    be1 = b_e1.reshape(1, 64)
    be2 = b_e2.reshape(1, 32)
    wia = W_imp[:32].reshape(1, 32)
    wib = W_imp[32:].reshape(1, 32)
    bi = b_imp.reshape(1, 1)

    out = pl.pallas_call(
        _chunk_kernel,
        grid=(B, K),
        in_specs=[
            pl.BlockSpec((1, TS, (S * DE) // 128, 128), lambda b, k: (b, k, 0, 0)),
            pl.BlockSpec((1, TS, HIDDEN), lambda b, k: (b, k, 0)),
            pl.BlockSpec((1, 1, 1, TS), lambda b, k: (b, k, 0, 0)),
            pl.BlockSpec((HIDDEN, 32), lambda b, k: (0, 0)),
            pl.BlockSpec((1, 32), lambda b, k: (0, 0)),
            pl.BlockSpec((1, 64), lambda b, k: (0, 0)),
            pl.BlockSpec((1, 64), lambda b, k: (0, 0)),
            pl.BlockSpec((64, 32), lambda b, k: (0, 0)),
            pl.BlockSpec((1, 32), lambda b, k: (0, 0)),
            pl.BlockSpec((1, 32), lambda b, k: (0, 0)),
            pl.BlockSpec((1, 32), lambda b, k: (0, 0)),
            pl.BlockSpec((1, 1), lambda b, k: (0, 0)),
        ],
        out_specs=pl.BlockSpec((1, 1, 128), lambda b, k: (b, 0, 0)),
        out_shape=jax.ShapeDtypeStruct((B, 1, 128), jnp.float32),
        compiler_params=pltpu.CompilerParams(
            dimension_semantics=("parallel", "arbitrary")),
    )(edge_r, node_features, mask_r, W_node, bn, W_e1, be1, W_e2, be2,
      wia, wib, bi)

    return (out[:, 0, 0], MAX_SEQ_LEN)


# speedup vs baseline: 3.2665x; 3.2665x over previous
import jax
import jax.numpy as jnp
from jax.experimental import pallas as pl
from jax.experimental.pallas import tpu as pltpu

B, S, DE, HIDDEN = 8, 1024, 4, 256
BASE_CHUNK = 64
MAX_SEQ_LEN = 512
TS = 256
K = S // TS


def _chunk_kernel(edge_ref, node_ref, mask_ref, wn_ref, bn_ref, we1_ref,
                  be1_ref, we2_ref, be2_ref, wia_ref, wib_ref, bi_ref,
                  out_ref):
    k = pl.program_id(1)

    e = edge_ref[0]
    r1 = jnp.sum(e, axis=1)
    rowsum = jnp.sum(r1, axis=1, keepdims=True)
    es = rowsum * (1.0 / (S * DE))

    node_enc = (jnp.dot(node_ref[0], wn_ref[...],
                        preferred_element_type=jnp.float32)
                + bn_ref[...])
    h = jnp.maximum(es * we1_ref[...] + be1_ref[...], 0.0)
    edge_enc = (jnp.dot(h, we2_ref[...],
                        preferred_element_type=jnp.float32)
                + be2_ref[...])

    logit = (jnp.sum(node_enc * wia_ref[...], axis=1, keepdims=True)
             + jnp.sum(edge_enc * wib_ref[...], axis=1, keepdims=True)
             + bi_ref[0, 0])
    sig = jax.nn.sigmoid(logit)
    m = mask_ref[0, 0]
    partial = jnp.dot(m, sig, preferred_element_type=jnp.float32)

    @pl.when(k == 0)
    def _init():
        out_ref[...] = jnp.zeros_like(out_ref)

    out_ref[...] += partial[0, 0]

    @pl.when(k == K - 1)
    def _finalize():
        acc = out_ref[...]
        cs = acc * (float(BASE_CHUNK) / float(S))
        cs = jnp.clip(cs, 32.0, 128.0)
        bad = (cs < 1.0) | ~jnp.isfinite(cs)
        out_ref[...] = jnp.where(bad, float(BASE_CHUNK), cs)


def kernel(node_features, edge_features, sequence_mask, W_node, b_node,
           W_e1, b_e1, W_e2, b_e2, W_imp, b_imp):
    edge_r = jnp.transpose(edge_features, (0, 1, 3, 2))
    mask_r = sequence_mask.reshape(B, K, 1, TS)
    bn = b_node.reshape(1, 32)
    be1 = b_e1.reshape(1, 64)
    be2 = b_e2.reshape(1, 32)
    wia = W_imp[:32].reshape(1, 32)
    wib = W_imp[32:].reshape(1, 32)
    bi = b_imp.reshape(1, 1)

    out = pl.pallas_call(
        _chunk_kernel,
        grid=(B, K),
        in_specs=[
            pl.BlockSpec((1, TS, DE, S), lambda b, k: (b, k, 0, 0)),
            pl.BlockSpec((1, TS, HIDDEN), lambda b, k: (b, k, 0)),
            pl.BlockSpec((1, 1, 1, TS), lambda b, k: (b, k, 0, 0)),
            pl.BlockSpec((HIDDEN, 32), lambda b, k: (0, 0)),
            pl.BlockSpec((1, 32), lambda b, k: (0, 0)),
            pl.BlockSpec((1, 64), lambda b, k: (0, 0)),
            pl.BlockSpec((1, 64), lambda b, k: (0, 0)),
            pl.BlockSpec((64, 32), lambda b, k: (0, 0)),
            pl.BlockSpec((1, 32), lambda b, k: (0, 0)),
            pl.BlockSpec((1, 32), lambda b, k: (0, 0)),
            pl.BlockSpec((1, 32), lambda b, k: (0, 0)),
            pl.BlockSpec((1, 1), lambda b, k: (0, 0)),
        ],
        out_specs=pl.BlockSpec((1, 1, 128), lambda b, k: (b, 0, 0)),
        out_shape=jax.ShapeDtypeStruct((B, 1, 128), jnp.float32),
        compiler_params=pltpu.CompilerParams(
            dimension_semantics=("parallel", "arbitrary")),
    )(edge_r, node_features, mask_r, W_node, bn, W_e1, be1, W_e2, be2,
      wia, wib, bi)

    return (out[:, 0, 0], MAX_SEQ_LEN)


# speedup vs baseline: 3.5103x; 1.0746x over previous
import jax
import jax.numpy as jnp
from jax.experimental import pallas as pl
from jax.experimental.pallas import tpu as pltpu

B, S, DE, HIDDEN = 8, 1024, 4, 256
BASE_CHUNK = 64
MAX_SEQ_LEN = 512
TS = 512
K = S // TS


def _chunk_kernel(edge_ref, node_ref, mask_ref, wn_ref, bn_ref, we1_ref,
                  be1_ref, we2_ref, be2_ref, wia_ref, wib_ref, bi_ref,
                  out_ref):
    k = pl.program_id(1)

    e = edge_ref[0]
    r1 = jnp.sum(e, axis=1)
    rowsum = jnp.sum(r1, axis=1, keepdims=True)
    es = rowsum * (1.0 / (S * DE))

    node_enc = (jnp.dot(node_ref[0], wn_ref[...],
                        preferred_element_type=jnp.float32)
                + bn_ref[...])
    h = jnp.maximum(es * we1_ref[...] + be1_ref[...], 0.0)
    edge_enc = (jnp.dot(h, we2_ref[...],
                        preferred_element_type=jnp.float32)
                + be2_ref[...])

    logit = (jnp.sum(node_enc * wia_ref[...], axis=1, keepdims=True)
             + jnp.sum(edge_enc * wib_ref[...], axis=1, keepdims=True)
             + bi_ref[0, 0])
    sig = jax.nn.sigmoid(logit)
    m = mask_ref[0, 0]
    partial = jnp.dot(m, sig, preferred_element_type=jnp.float32)

    @pl.when(k == 0)
    def _init():
        out_ref[...] = jnp.zeros_like(out_ref)

    out_ref[...] += partial[0, 0]

    @pl.when(k == K - 1)
    def _finalize():
        acc = out_ref[...]
        cs = acc * (float(BASE_CHUNK) / float(S))
        cs = jnp.clip(cs, 32.0, 128.0)
        bad = (cs < 1.0) | ~jnp.isfinite(cs)
        out_ref[...] = jnp.where(bad, float(BASE_CHUNK), cs)


def kernel(node_features, edge_features, sequence_mask, W_node, b_node,
           W_e1, b_e1, W_e2, b_e2, W_imp, b_imp):
    edge_r = jnp.transpose(edge_features, (0, 1, 3, 2))
    mask_r = sequence_mask.reshape(B, K, 1, TS)
    bn = b_node.reshape(1, 32)
    be1 = b_e1.reshape(1, 64)
    be2 = b_e2.reshape(1, 32)
    wia = W_imp[:32].reshape(1, 32)
    wib = W_imp[32:].reshape(1, 32)
    bi = b_imp.reshape(1, 1)

    out = pl.pallas_call(
        _chunk_kernel,
        grid=(B, K),
        in_specs=[
            pl.BlockSpec((1, TS, DE, S), lambda b, k: (b, k, 0, 0)),
            pl.BlockSpec((1, TS, HIDDEN), lambda b, k: (b, k, 0)),
            pl.BlockSpec((1, 1, 1, TS), lambda b, k: (b, k, 0, 0)),
            pl.BlockSpec((HIDDEN, 32), lambda b, k: (0, 0)),
            pl.BlockSpec((1, 32), lambda b, k: (0, 0)),
            pl.BlockSpec((1, 64), lambda b, k: (0, 0)),
            pl.BlockSpec((1, 64), lambda b, k: (0, 0)),
            pl.BlockSpec((64, 32), lambda b, k: (0, 0)),
            pl.BlockSpec((1, 32), lambda b, k: (0, 0)),
            pl.BlockSpec((1, 32), lambda b, k: (0, 0)),
            pl.BlockSpec((1, 32), lambda b, k: (0, 0)),
            pl.BlockSpec((1, 1), lambda b, k: (0, 0)),
        ],
        out_specs=pl.BlockSpec((1, 1, 128), lambda b, k: (b, 0, 0)),
        out_shape=jax.ShapeDtypeStruct((B, 1, 128), jnp.float32),
        compiler_params=pltpu.CompilerParams(
            dimension_semantics=("parallel", "arbitrary")),
    )(edge_r, node_features, mask_r, W_node, bn, W_e1, be1, W_e2, be2,
      wia, wib, bi)

    return (out[:, 0, 0], MAX_SEQ_LEN)


# speedup vs baseline: 3.5542x; 1.0125x over previous
import jax
import jax.numpy as jnp
from jax.experimental import pallas as pl
from jax.experimental.pallas import tpu as pltpu

B, S, DE, HIDDEN = 8, 1024, 4, 256
BASE_CHUNK = 64
MAX_SEQ_LEN = 512
TS = 1024
K = S // TS


def _chunk_kernel(edge_ref, node_ref, mask_ref, wn_ref, bn_ref, we1_ref,
                  be1_ref, we2_ref, be2_ref, wia_ref, wib_ref, bi_ref,
                  out_ref):
    k = pl.program_id(1)

    e = edge_ref[0]
    r1 = jnp.sum(e, axis=1)
    rowsum = jnp.sum(r1, axis=1, keepdims=True)
    es = rowsum * (1.0 / (S * DE))

    node_enc = (jnp.dot(node_ref[0], wn_ref[...],
                        preferred_element_type=jnp.float32)
                + bn_ref[...])
    h = jnp.maximum(es * we1_ref[...] + be1_ref[...], 0.0)
    edge_enc = (jnp.dot(h, we2_ref[...],
                        preferred_element_type=jnp.float32)
                + be2_ref[...])

    logit = (jnp.sum(node_enc * wia_ref[...], axis=1, keepdims=True)
             + jnp.sum(edge_enc * wib_ref[...], axis=1, keepdims=True)
             + bi_ref[0, 0])
    sig = jax.nn.sigmoid(logit)
    m = mask_ref[0, 0]
    partial = jnp.dot(m, sig, preferred_element_type=jnp.float32)

    @pl.when(k == 0)
    def _init():
        out_ref[...] = jnp.zeros_like(out_ref)

    out_ref[...] += partial[0, 0]

    @pl.when(k == K - 1)
    def _finalize():
        acc = out_ref[...]
        cs = acc * (float(BASE_CHUNK) / float(S))
        cs = jnp.clip(cs, 32.0, 128.0)
        bad = (cs < 1.0) | ~jnp.isfinite(cs)
        out_ref[...] = jnp.where(bad, float(BASE_CHUNK), cs)


def kernel(node_features, edge_features, sequence_mask, W_node, b_node,
           W_e1, b_e1, W_e2, b_e2, W_imp, b_imp):
    edge_r = jnp.transpose(edge_features, (0, 1, 3, 2))
    mask_r = sequence_mask.reshape(B, K, 1, TS)
    bn = b_node.reshape(1, 32)
    be1 = b_e1.reshape(1, 64)
    be2 = b_e2.reshape(1, 32)
    wia = W_imp[:32].reshape(1, 32)
    wib = W_imp[32:].reshape(1, 32)
    bi = b_imp.reshape(1, 1)

    out = pl.pallas_call(
        _chunk_kernel,
        grid=(B, K),
        in_specs=[
            pl.BlockSpec((1, TS, DE, S), lambda b, k: (b, k, 0, 0)),
            pl.BlockSpec((1, TS, HIDDEN), lambda b, k: (b, k, 0)),
            pl.BlockSpec((1, 1, 1, TS), lambda b, k: (b, k, 0, 0)),
            pl.BlockSpec((HIDDEN, 32), lambda b, k: (0, 0)),
            pl.BlockSpec((1, 32), lambda b, k: (0, 0)),
            pl.BlockSpec((1, 64), lambda b, k: (0, 0)),
            pl.BlockSpec((1, 64), lambda b, k: (0, 0)),
            pl.BlockSpec((64, 32), lambda b, k: (0, 0)),
            pl.BlockSpec((1, 32), lambda b, k: (0, 0)),
            pl.BlockSpec((1, 32), lambda b, k: (0, 0)),
            pl.BlockSpec((1, 32), lambda b, k: (0, 0)),
            pl.BlockSpec((1, 1), lambda b, k: (0, 0)),
        ],
        out_specs=pl.BlockSpec((1, 1, 128), lambda b, k: (b, 0, 0)),
        out_shape=jax.ShapeDtypeStruct((B, 1, 128), jnp.float32),
        compiler_params=pltpu.CompilerParams(
            dimension_semantics=("parallel", "arbitrary")),
    )(edge_r, node_features, mask_r, W_node, bn, W_e1, be1, W_e2, be2,
      wia, wib, bi)

    return (out[:, 0, 0], MAX_SEQ_LEN)


# speedup vs baseline: 6.0346x; 1.6979x over previous
import jax
import jax.numpy as jnp
from jax.experimental import pallas as pl
from jax.experimental.pallas import tpu as pltpu

B, S, DE, HIDDEN = 8, 1024, 4, 256
BASE_CHUNK = 64
MAX_SEQ_LEN = 512
TS = 1024
K = S // TS


def _chunk_kernel(edge_ref, node_ref, mask_ref, wn_ref, bn_ref, we1_ref,
                  be1_ref, we2_ref, be2_ref, wia_ref, wib_ref, bi_ref,
                  out_ref):
    k = pl.program_id(1)

    e = edge_ref[0]
    r1 = jnp.sum(e, axis=2)
    rowsum = jnp.sum(r1, axis=1, keepdims=True)
    es = rowsum * (1.0 / (S * DE))

    node_enc = (jnp.dot(node_ref[0], wn_ref[...],
                        preferred_element_type=jnp.float32)
                + bn_ref[...])
    h = jnp.maximum(es * we1_ref[...] + be1_ref[...], 0.0)
    edge_enc = (jnp.dot(h, we2_ref[...],
                        preferred_element_type=jnp.float32)
                + be2_ref[...])

    logit = (jnp.sum(node_enc * wia_ref[...], axis=1, keepdims=True)
             + jnp.sum(edge_enc * wib_ref[...], axis=1, keepdims=True)
             + bi_ref[0, 0])
    sig = jax.nn.sigmoid(logit)
    m = mask_ref[0, 0]
    partial = jnp.dot(m, sig, preferred_element_type=jnp.float32)

    @pl.when(k == 0)
    def _init():
        out_ref[...] = jnp.zeros_like(out_ref)

    out_ref[...] += partial[0, 0]

    @pl.when(k == K - 1)
    def _finalize():
        acc = out_ref[...]
        cs = acc * (float(BASE_CHUNK) / float(S))
        cs = jnp.clip(cs, 32.0, 128.0)
        bad = (cs < 1.0) | ~jnp.isfinite(cs)
        out_ref[...] = jnp.where(bad, float(BASE_CHUNK), cs)


def kernel(node_features, edge_features, sequence_mask, W_node, b_node,
           W_e1, b_e1, W_e2, b_e2, W_imp, b_imp):
    edge_r = jnp.transpose(edge_features, (0, 1, 3, 2))
    mask_r = sequence_mask.reshape(B, K, 1, TS)
    bn = b_node.reshape(1, 32)
    be1 = b_e1.reshape(1, 64)
    be2 = b_e2.reshape(1, 32)
    wia = W_imp[:32].reshape(1, 32)
    wib = W_imp[32:].reshape(1, 32)
    bi = b_imp.reshape(1, 1)

    out = pl.pallas_call(
        _chunk_kernel,
        grid=(B, K),
        in_specs=[
            pl.BlockSpec((1, TS, DE, S), lambda b, k: (b, k, 0, 0)),
            pl.BlockSpec((1, TS, HIDDEN), lambda b, k: (b, k, 0)),
            pl.BlockSpec((1, 1, 1, TS), lambda b, k: (b, k, 0, 0)),
            pl.BlockSpec((HIDDEN, 32), lambda b, k: (0, 0)),
            pl.BlockSpec((1, 32), lambda b, k: (0, 0)),
            pl.BlockSpec((1, 64), lambda b, k: (0, 0)),
            pl.BlockSpec((1, 64), lambda b, k: (0, 0)),
            pl.BlockSpec((64, 32), lambda b, k: (0, 0)),
            pl.BlockSpec((1, 32), lambda b, k: (0, 0)),
            pl.BlockSpec((1, 32), lambda b, k: (0, 0)),
            pl.BlockSpec((1, 32), lambda b, k: (0, 0)),
            pl.BlockSpec((1, 1), lambda b, k: (0, 0)),
        ],
        out_specs=pl.BlockSpec((1, 1, 128), lambda b, k: (b, 0, 0)),
        out_shape=jax.ShapeDtypeStruct((B, 1, 128), jnp.float32),
        compiler_params=pltpu.CompilerParams(
            dimension_semantics=("parallel", "arbitrary")),
    )(edge_r, node_features, mask_r, W_node, bn, W_e1, be1, W_e2, be2,
      wia, wib, bi)

    return (out[:, 0, 0], MAX_SEQ_LEN)
